# Initial kernel scaffold; baseline (speedup 1.0000x reference)
import jax
import jax.numpy as jnp
from jax.experimental import pallas as pl


def kernel(features, edge_index, W1, W2):
    raise NotImplementedError("write your pallas kernel here")


# speedup vs baseline: 12.4804x; 12.4804x over previous
import functools

import jax
import jax.numpy as jnp
from jax import lax
from jax.experimental import pallas as pl
from jax.experimental.pallas import tpu as pltpu
from jax.experimental.pallas import tpu_sc as plsc

N_NODES = 10000
N_EDGES = 320000
D_IN = 128
D_H = 16

NC = 2
NS = 16
NW = NC * NS

N_PAD = 10240
CHUNK = 128
KJ = (N_EDGES + NW * CHUNK - 1) // (NW * CHUNK)
E_PAD = NW * CHUNK * KJ
ROWS_PER_TILE = N_PAD // NS

@functools.cache
def _build_spmm_sc():
    mesh = plsc.VectorSubcoreMesh(
        core_axis_name="c", subcore_axis_name="s", num_cores=NC, num_subcores=NS
    )
    return pl.kernel(
        _spmm_body,
        out_type=jax.ShapeDtypeStruct((NC, N_PAD, D_H), jnp.float32),
        mesh=mesh,
        scratch_types=[
            pltpu.VMEM((KJ, CHUNK), jnp.int32),
            pltpu.VMEM((KJ, CHUNK), jnp.int32),
            pltpu.VMEM((2, CHUNK, D_H), jnp.float32),
            pltpu.VMEM((ROWS_PER_TILE, D_H), jnp.float32),
            pltpu.VMEM_SHARED((N_PAD, D_H), jnp.float32),
            pltpu.SemaphoreType.DMA,
        ],
        compiler_params=pltpu.CompilerParams(use_tc_tiling_on_sc=False),
    )


def _spmm_body(x_hbm, src_hbm, dst_hbm, out_hbm, src_v, dst_v, rows_v, zbuf, acc, sem):
    cid = lax.axis_index("c")
    sid = lax.axis_index("s")
    wid = sid * NC + cid

    def _zero(i, carry):
        zbuf[i, :] = jnp.zeros((D_H,), jnp.float32)
        return carry

    lax.fori_loop(0, ROWS_PER_TILE, _zero, 0)
    pltpu.sync_copy(zbuf, acc.at[pl.ds(sid * ROWS_PER_TILE, ROWS_PER_TILE)])

    pltpu.sync_copy(src_hbm.at[wid], src_v)
    pltpu.sync_copy(dst_hbm.at[wid], dst_v)
    plsc.subcore_barrier()

    pltpu.async_copy(x_hbm.at[src_v.at[0]], rows_v.at[0], sem)

    def _body(j, carry):
        buf = lax.rem(j, 2)
        pltpu.make_async_copy(x_hbm.at[src_v.at[j]], rows_v.at[buf], sem).wait()
        pltpu.async_copy(x_hbm.at[src_v.at[j + 1]], rows_v.at[1 - buf], sem)
        pltpu.sync_copy(rows_v.at[buf], acc.at[dst_v.at[j]], add=True)
        return carry

    lax.fori_loop(0, KJ - 1, _body, 0)
    last = KJ - 1
    buf = lax.rem(last, 2)
    pltpu.make_async_copy(x_hbm.at[src_v.at[last]], rows_v.at[buf], sem).wait()
    pltpu.sync_copy(rows_v.at[buf], acc.at[dst_v.at[last]], add=True)
    plsc.subcore_barrier()

    pltpu.sync_copy(
        acc.at[pl.ds(sid * ROWS_PER_TILE, ROWS_PER_TILE)],
        out_hbm.at[cid, pl.ds(sid * ROWS_PER_TILE, ROWS_PER_TILE)],
    )


def _mm1_body(f_ref, w_ref, o_ref):
    o_ref[...] = jnp.dot(f_ref[...], w_ref[...], preferred_element_type=jnp.float32)


def _mm2_body(p0_ref, p1_ref, w_ref, o_ref):
    h = jnp.maximum(p0_ref[...] + p1_ref[...], 0.0)
    o_ref[...] = jnp.dot(h, w_ref[...], preferred_element_type=jnp.float32)


def _add_body(p0_ref, p1_ref, o_ref):
    o_ref[...] = p0_ref[...] + p1_ref[...]


def kernel(features, edge_index, W1, W2):
    f_pad = jnp.pad(features, ((0, N_PAD - N_NODES), (0, 0)))
    w2_pad = jnp.pad(W2, ((0, 0), (0, D_H - W2.shape[1])))

    src = edge_index[0].astype(jnp.int32)
    dst = edge_index[1].astype(jnp.int32)
    pad_n = E_PAD - N_EDGES
    fill = jnp.full((pad_n,), N_NODES, jnp.int32)
    srcb = jnp.concatenate([src, fill]).reshape(NW, KJ, CHUNK)
    dstb = jnp.concatenate([dst, fill]).reshape(NW, KJ, CHUNK)

    x1 = pl.pallas_call(
        _mm1_body,
        grid=(8,),
        in_specs=[
            pl.BlockSpec((N_PAD // 8, D_IN), lambda i: (i, 0)),
            pl.BlockSpec((D_IN, D_H), lambda i: (0, 0)),
        ],
        out_specs=pl.BlockSpec((N_PAD // 8, D_H), lambda i: (i, 0)),
        out_shape=jax.ShapeDtypeStruct((N_PAD, D_H), jnp.float32),
    )(f_pad, W1)

    spmm = _build_spmm_sc()
    p = spmm(x1, srcb, dstb)

    x2 = pl.pallas_call(
        _mm2_body,
        grid=(4,),
        in_specs=[
            pl.BlockSpec((N_PAD // 4, D_H), lambda i: (i, 0)),
            pl.BlockSpec((N_PAD // 4, D_H), lambda i: (i, 0)),
            pl.BlockSpec((D_H, D_H), lambda i: (0, 0)),
        ],
        out_specs=pl.BlockSpec((N_PAD // 4, D_H), lambda i: (i, 0)),
        out_shape=jax.ShapeDtypeStruct((N_PAD, D_H), jnp.float32),
    )(p[0], p[1], w2_pad)

    q = spmm(x2, srcb, dstb)

    out = pl.pallas_call(
        _add_body,
        grid=(4,),
        in_specs=[
            pl.BlockSpec((N_PAD // 4, D_H), lambda i: (i, 0)),
            pl.BlockSpec((N_PAD // 4, D_H), lambda i: (i, 0)),
        ],
        out_specs=pl.BlockSpec((N_PAD // 4, D_H), lambda i: (i, 0)),
        out_shape=jax.ShapeDtypeStruct((N_PAD, D_H), jnp.float32),
    )(q[0], q[1])

    return out[:N_NODES, : W2.shape[1]]


# speedup vs baseline: 14.6670x; 1.1752x over previous
import functools

import jax
import jax.numpy as jnp
from jax import lax
from jax.experimental import pallas as pl
from jax.experimental.pallas import tpu as pltpu
from jax.experimental.pallas import tpu_sc as plsc

N_NODES = 10000
D_IN = 128
D_H = 16
D_OUT = 7

NC = 2
NS = 16
NW = NC * NS

N_PAD = 10240
CHUNK = 128
ROWS_PER_TILE = N_PAD // NS


def _spmm_body(d, kj, x_hbm, eb_hbm, out_hbm, src_v, dst_v, rows_v, zbuf, acc, sem):
    cid = lax.axis_index("c")
    sid = lax.axis_index("s")
    wid = sid * NC + cid

    def _zero(i, carry):
        zbuf[i, :] = jnp.zeros((d,), jnp.float32)
        return carry

    lax.fori_loop(0, ROWS_PER_TILE, _zero, 0)
    pltpu.sync_copy(zbuf, acc.at[pl.ds(sid * ROWS_PER_TILE, ROWS_PER_TILE)])

    pltpu.sync_copy(eb_hbm.at[0, wid], src_v)
    pltpu.sync_copy(eb_hbm.at[1, wid], dst_v)
    plsc.subcore_barrier()

    pltpu.async_copy(x_hbm.at[src_v.at[0]], rows_v.at[0], sem)

    def _chunk(j, carry):
        buf = lax.rem(j, 2)
        pltpu.make_async_copy(x_hbm.at[src_v.at[j]], rows_v.at[buf], sem).wait()
        pltpu.async_copy(x_hbm.at[src_v.at[j + 1]], rows_v.at[1 - buf], sem)
        pltpu.sync_copy(rows_v.at[buf], acc.at[dst_v.at[j]], add=True)
        return carry

    lax.fori_loop(0, kj - 1, _chunk, 0)
    last = kj - 1
    buf = lax.rem(last, 2)
    pltpu.make_async_copy(x_hbm.at[src_v.at[last]], rows_v.at[buf], sem).wait()
    pltpu.sync_copy(rows_v.at[buf], acc.at[dst_v.at[last]], add=True)
    plsc.subcore_barrier()

    pltpu.sync_copy(
        acc.at[pl.ds(sid * ROWS_PER_TILE, ROWS_PER_TILE)],
        out_hbm.at[cid, pl.ds(sid * ROWS_PER_TILE, ROWS_PER_TILE)],
    )


@functools.cache
def _build_spmm_sc(d, kj):
    mesh = plsc.VectorSubcoreMesh(
        core_axis_name="c", subcore_axis_name="s", num_cores=NC, num_subcores=NS
    )
    return pl.kernel(
        functools.partial(_spmm_body, d, kj),
        out_type=jax.ShapeDtypeStruct((NC, N_PAD, d), jnp.float32),
        mesh=mesh,
        scratch_types=[
            pltpu.VMEM((kj, CHUNK), jnp.int32),
            pltpu.VMEM((kj, CHUNK), jnp.int32),
            pltpu.VMEM((2, CHUNK, d), jnp.float32),
            pltpu.VMEM((ROWS_PER_TILE, d), jnp.float32),
            pltpu.VMEM_SHARED((N_PAD, d), jnp.float32),
            pltpu.SemaphoreType.DMA,
        ],
        compiler_params=pltpu.CompilerParams(use_tc_tiling_on_sc=False),
    )


def _mm1_body(f_ref, w_ref, o_ref):
    o_ref[...] = jnp.dot(f_ref[...], w_ref[...], preferred_element_type=jnp.float32)


def _mm2_body(p0_ref, p1_ref, w_ref, o_ref):
    h = jnp.maximum(p0_ref[0] + p1_ref[0], 0.0)
    w = jnp.pad(w_ref[...], ((0, 0), (0, 8 - D_OUT)))
    o_ref[...] = jnp.dot(h, w, preferred_element_type=jnp.float32)


def _combine_body(q0_ref, q1_ref, o_ref):
    o_ref[...] = (q0_ref[0] + q1_ref[0])[:, :D_OUT]


def kernel(features, edge_index, W1, W2):
    e = edge_index.shape[1]
    kj = (e + NW * CHUNK - 1) // (NW * CHUNK)
    e_pad = NW * CHUNK * kj
    eb = jnp.pad(
        edge_index.astype(jnp.int32),
        ((0, 0), (0, e_pad - e)),
        constant_values=N_NODES,
    ).reshape(2, NW, kj, CHUNK)

    x1 = pl.pallas_call(
        _mm1_body,
        grid=(8,),
        in_specs=[
            pl.BlockSpec((N_PAD // 8, D_IN), lambda i: (i, 0)),
            pl.BlockSpec((D_IN, D_H), lambda i: (0, 0)),
        ],
        out_specs=pl.BlockSpec((N_PAD // 8, D_H), lambda i: (i, 0)),
        out_shape=jax.ShapeDtypeStruct((N_PAD, D_H), jnp.float32),
    )(features, W1)

    p = _build_spmm_sc(D_H, kj)(x1, eb)

    x2 = pl.pallas_call(
        _mm2_body,
        grid=(4,),
        in_specs=[
            pl.BlockSpec((1, N_PAD // 4, D_H), lambda i: (0, i, 0)),
            pl.BlockSpec((1, N_PAD // 4, D_H), lambda i: (1, i, 0)),
            pl.BlockSpec((D_H, D_OUT), lambda i: (0, 0)),
        ],
        out_specs=pl.BlockSpec((N_PAD // 4, 8), lambda i: (i, 0)),
        out_shape=jax.ShapeDtypeStruct((N_PAD, 8), jnp.float32),
    )(p, p, W2)

    q = _build_spmm_sc(8, kj)(x2, eb)

    out = pl.pallas_call(
        _combine_body,
        grid=(5,),
        in_specs=[
            pl.BlockSpec((1, N_NODES // 5, 8), lambda i: (0, i, 0)),
            pl.BlockSpec((1, N_NODES // 5, 8), lambda i: (1, i, 0)),
        ],
        out_specs=pl.BlockSpec((N_NODES // 5, D_OUT), lambda i: (i, 0)),
        out_shape=jax.ShapeDtypeStruct((N_NODES, D_OUT), jnp.float32),
    )(q, q)

    return out


# speedup vs baseline: 27.5938x; 1.8814x over previous
import functools

import jax
import jax.numpy as jnp
from jax import lax
from jax.experimental import pallas as pl
from jax.experimental.pallas import tpu as pltpu
from jax.experimental.pallas import tpu_sc as plsc

N_NODES = 10000
D_IN = 128
D_H = 16
D_OUT = 7

NC = 2
NS = 16
NW = NC * NS

N_PAD = 10240
CHUNK = 128
ROWS_PER_TILE = N_PAD // NS
L = 16


def _aggregate(d, kj, eb_hbm, out_hbm, src_v, dst_v, rows_v, xs, acc, sem,
               cid, sid, wid):
    pltpu.sync_copy(eb_hbm.at[0, wid], src_v)
    pltpu.sync_copy(eb_hbm.at[1, wid], dst_v)
    plsc.subcore_barrier()

    pltpu.async_copy(xs.at[src_v.at[0]], rows_v.at[0], sem)

    def _chunk(j, carry):
        buf = lax.rem(j, 2)
        pltpu.make_async_copy(xs.at[src_v.at[j]], rows_v.at[buf], sem).wait()
        pltpu.async_copy(xs.at[src_v.at[j + 1]], rows_v.at[1 - buf], sem)
        pltpu.sync_copy(rows_v.at[buf], acc.at[dst_v.at[j]], add=True)
        return carry

    lax.fori_loop(0, kj - 1, _chunk, 0)
    last = kj - 1
    buf = lax.rem(last, 2)
    pltpu.make_async_copy(xs.at[src_v.at[last]], rows_v.at[buf], sem).wait()
    pltpu.sync_copy(rows_v.at[buf], acc.at[dst_v.at[last]], add=True)
    plsc.subcore_barrier()

    pltpu.sync_copy(
        acc.at[pl.ds(sid * ROWS_PER_TILE, ROWS_PER_TILE)],
        out_hbm.at[cid, pl.ds(sid * ROWS_PER_TILE, ROWS_PER_TILE)],
    )


def _zero_acc(d, zbuf, acc, sid):
    if d == L:
        def _zero(i, carry):
            zbuf[i, :] = jnp.zeros((L,), jnp.float32)
            return carry

        lax.fori_loop(0, ROWS_PER_TILE, _zero, 0)
    else:
        zf = jnp.zeros((L,), jnp.float32)
        zi = jnp.zeros((L,), jnp.int32)
        riota = lax.iota(jnp.int32, L)

        def _zero(g, carry):
            rows = g * L + riota
            for j in range(d):
                plsc.store_scatter(zbuf, [rows, zi + j], zf)
            return carry

        lax.fori_loop(0, ROWS_PER_TILE // L, _zero, 0)
    pltpu.sync_copy(zbuf, acc.at[pl.ds(sid * ROWS_PER_TILE, ROWS_PER_TILE)])


def _spmm1_body(kj, x_hbm, eb_hbm, out_hbm,
                src_v, dst_v, rows_v, zbuf, xs, acc, sem):
    cid = lax.axis_index("c")
    sid = lax.axis_index("s")
    wid = sid * NC + cid

    pltpu.sync_copy(
        x_hbm.at[pl.ds(sid * ROWS_PER_TILE, ROWS_PER_TILE)],
        xs.at[pl.ds(sid * ROWS_PER_TILE, ROWS_PER_TILE)],
    )
    _zero_acc(D_H, zbuf, acc, sid)
    _aggregate(D_H, kj, eb_hbm, out_hbm, src_v, dst_v, rows_v, xs, acc, sem,
               cid, sid, wid)


def _spmm2_body(kj, p_hbm, eb_hbm, w_hbm, out_hbm,
                src_v, dst_v, rows_v, zbuf, pv, x2v, w_v, xs, acc, sem):
    cid = lax.axis_index("c")
    sid = lax.axis_index("s")
    wid = sid * NC + cid
    base = sid * ROWS_PER_TILE

    pltpu.sync_copy(w_hbm, w_v)
    pltpu.sync_copy(p_hbm.at[0, pl.ds(base, ROWS_PER_TILE)], pv.at[0])
    pltpu.sync_copy(p_hbm.at[1, pl.ds(base, ROWS_PER_TILE)], pv.at[1])

    def _hrow(i, carry):
        pv[0, i, :] = jnp.maximum(pv[0, i, :] + pv[1, i, :], 0.0)
        return carry

    lax.fori_loop(0, ROWS_PER_TILE, _hrow, 0)

    zero_i = jnp.zeros((L,), jnp.int32)
    riota = lax.iota(jnp.int32, L)
    wregs = [w_v[pl.ds(t * L, L)] for t in range(8 * D_H // L)]

    def _wscal(k, j):
        flat = k * 8 + j
        return wregs[flat // L][flat % L]

    def _group(g, carry):
        rows = g * L + riota
        cols = [None] * D_H
        for k in range(D_H):
            cols[k] = plsc.load_gather(pv, [zero_i, rows, zero_i + k])
        for j in range(8):
            o = cols[0] * _wscal(0, j)
            for k in range(1, D_H):
                o = o + cols[k] * _wscal(k, j)
            plsc.store_scatter(x2v, [rows, zero_i + j], o)
        return carry

    lax.fori_loop(0, ROWS_PER_TILE // L, _group, 0)
    pltpu.sync_copy(x2v, xs.at[pl.ds(base, ROWS_PER_TILE)])

    _zero_acc(8, zbuf, acc, sid)
    _aggregate(8, kj, eb_hbm, out_hbm, src_v, dst_v, rows_v, xs, acc, sem,
               cid, sid, wid)


@functools.cache
def _build_spmm1(kj):
    mesh = plsc.VectorSubcoreMesh(
        core_axis_name="c", subcore_axis_name="s", num_cores=NC, num_subcores=NS
    )
    return pl.kernel(
        functools.partial(_spmm1_body, kj),
        out_type=jax.ShapeDtypeStruct((NC, N_PAD, D_H), jnp.float32),
        mesh=mesh,
        scratch_types=[
            pltpu.VMEM((kj, CHUNK), jnp.int32),
            pltpu.VMEM((kj, CHUNK), jnp.int32),
            pltpu.VMEM((2, CHUNK, D_H), jnp.float32),
            pltpu.VMEM((ROWS_PER_TILE, D_H), jnp.float32),
            pltpu.VMEM_SHARED((N_PAD, D_H), jnp.float32),
            pltpu.VMEM_SHARED((N_PAD, D_H), jnp.float32),
            pltpu.SemaphoreType.DMA,
        ],
        compiler_params=pltpu.CompilerParams(use_tc_tiling_on_sc=False),
    )


@functools.cache
def _build_spmm2(kj):
    mesh = plsc.VectorSubcoreMesh(
        core_axis_name="c", subcore_axis_name="s", num_cores=NC, num_subcores=NS
    )
    return pl.kernel(
        functools.partial(_spmm2_body, kj),
        out_type=jax.ShapeDtypeStruct((NC, N_PAD, 8), jnp.float32),
        mesh=mesh,
        scratch_types=[
            pltpu.VMEM((kj, CHUNK), jnp.int32),
            pltpu.VMEM((kj, CHUNK), jnp.int32),
            pltpu.VMEM((2, CHUNK, 8), jnp.float32),
            pltpu.VMEM((ROWS_PER_TILE, 8), jnp.float32),
            pltpu.VMEM((2, ROWS_PER_TILE, D_H), jnp.float32),
            pltpu.VMEM((ROWS_PER_TILE, 8), jnp.float32),
            pltpu.VMEM((8 * D_H,), jnp.float32),
            pltpu.VMEM_SHARED((N_PAD, 8), jnp.float32),
            pltpu.VMEM_SHARED((N_PAD, 8), jnp.float32),
            pltpu.SemaphoreType.DMA,
        ],
        compiler_params=pltpu.CompilerParams(
            use_tc_tiling_on_sc=False, needs_layout_passes=False
        ),
    )


def _mm1_body(f_ref, w1_ref, o_ref):
    o_ref[...] = jnp.dot(f_ref[...], w1_ref[...],
                         preferred_element_type=jnp.float32)


def _combine_body(q0_ref, q1_ref, o_ref):
    o_ref[...] = (q0_ref[0] + q1_ref[0])[:, :D_OUT]


def kernel(features, edge_index, W1, W2):
    e = edge_index.shape[1]
    kj = (e + NW * CHUNK - 1) // (NW * CHUNK)
    e_pad = NW * CHUNK * kj
    eb = jnp.pad(
        edge_index.astype(jnp.int32),
        ((0, 0), (0, e_pad - e)),
        constant_values=N_NODES,
    ).reshape(2, NW, kj, CHUNK)

    w2f = jnp.pad(W2, ((0, 0), (0, 8 - D_OUT))).reshape(8 * D_H)

    x1 = pl.pallas_call(
        _mm1_body,
        grid=(8,),
        in_specs=[
            pl.BlockSpec((N_PAD // 8, D_IN), lambda i: (i, 0)),
            pl.BlockSpec((D_IN, D_H), lambda i: (0, 0)),
        ],
        out_specs=pl.BlockSpec((N_PAD // 8, D_H), lambda i: (i, 0)),
        out_shape=jax.ShapeDtypeStruct((N_PAD, D_H), jnp.float32),
    )(features, W1)

    p = _build_spmm1(kj)(x1, eb)

    q = _build_spmm2(kj)(p, eb, w2f)

    out = pl.pallas_call(
        _combine_body,
        grid=(5,),
        in_specs=[
            pl.BlockSpec((1, N_NODES // 5, 8), lambda i: (0, i, 0)),
            pl.BlockSpec((1, N_NODES // 5, 8), lambda i: (1, i, 0)),
        ],
        out_specs=pl.BlockSpec((N_NODES // 5, D_OUT), lambda i: (i, 0)),
        out_shape=jax.ShapeDtypeStruct((N_NODES, D_OUT), jnp.float32),
    )(q, q)

    return out


# speedup vs baseline: 28.9479x; 1.0491x over previous
import functools

import jax
import jax.numpy as jnp
from jax import lax
from jax.experimental import pallas as pl
from jax.experimental.pallas import tpu as pltpu
from jax.experimental.pallas import tpu_sc as plsc

N_NODES = 10000
D_IN = 128
D_H = 16
D_OUT = 7

NC = 2
NS = 16
NW = NC * NS

N_PAD = 10240
CHUNK = 128
ROWS_PER_TILE = N_PAD // NS
L = 16


def _aggregate(d, kj, eb_hbm, out_hbm, src_v, dst_v, rows_v, xs, acc, sem,
               cid, sid, wid):
    pltpu.sync_copy(eb_hbm.at[0, wid], src_v)
    pltpu.sync_copy(eb_hbm.at[1, wid], dst_v)
    plsc.subcore_barrier()

    pltpu.async_copy(xs.at[src_v.at[0]], rows_v.at[0], sem)

    def _chunk(j, carry):
        buf = lax.rem(j, 2)
        pltpu.make_async_copy(xs.at[src_v.at[j]], rows_v.at[buf], sem).wait()
        pltpu.async_copy(xs.at[src_v.at[j + 1]], rows_v.at[1 - buf], sem)
        pltpu.sync_copy(rows_v.at[buf], acc.at[dst_v.at[j]], add=True)
        return carry

    lax.fori_loop(0, kj - 1, _chunk, 0)
    last = kj - 1
    buf = lax.rem(last, 2)
    pltpu.make_async_copy(xs.at[src_v.at[last]], rows_v.at[buf], sem).wait()
    pltpu.sync_copy(rows_v.at[buf], acc.at[dst_v.at[last]], add=True)
    plsc.subcore_barrier()

    pltpu.sync_copy(
        acc.at[pl.ds(sid * ROWS_PER_TILE, ROWS_PER_TILE)],
        out_hbm.at[cid, pl.ds(sid * ROWS_PER_TILE, ROWS_PER_TILE)],
    )


def _zero_acc(d, zbuf, acc, sid):
    if d == L:
        def _zero(i, carry):
            zbuf[i, :] = jnp.zeros((L,), jnp.float32)
            return carry

        lax.fori_loop(0, ROWS_PER_TILE, _zero, 0)
    else:
        zf = jnp.zeros((L,), jnp.float32)
        zi = jnp.zeros((L,), jnp.int32)
        riota = lax.iota(jnp.int32, L)

        def _zero(g, carry):
            rows = g * L + riota
            for j in range(d):
                plsc.store_scatter(zbuf, [rows, zi + j], zf)
            return carry

        lax.fori_loop(0, ROWS_PER_TILE // L, _zero, 0)
    pltpu.sync_copy(zbuf, acc.at[pl.ds(sid * ROWS_PER_TILE, ROWS_PER_TILE)])


def _spmm1_body(kj, x_hbm, eb_hbm, out_hbm,
                src_v, dst_v, rows_v, zbuf, xs, acc, sem):
    cid = lax.axis_index("c")
    sid = lax.axis_index("s")
    wid = sid * NC + cid

    pltpu.sync_copy(
        x_hbm.at[pl.ds(sid * ROWS_PER_TILE, ROWS_PER_TILE)],
        xs.at[pl.ds(sid * ROWS_PER_TILE, ROWS_PER_TILE)],
    )
    _zero_acc(D_H, zbuf, acc, sid)
    _aggregate(D_H, kj, eb_hbm, out_hbm, src_v, dst_v, rows_v, xs, acc, sem,
               cid, sid, wid)


def _spmm2_body(kj, p_hbm, eb_hbm, w_hbm, out_hbm,
                src_v, dst_v, rows_v, zbuf, pv, x2v, w_v, xs, acc, sem):
    cid = lax.axis_index("c")
    sid = lax.axis_index("s")
    wid = sid * NC + cid
    base = sid * ROWS_PER_TILE

    pltpu.sync_copy(w_hbm, w_v)
    pltpu.sync_copy(p_hbm.at[0, pl.ds(base, ROWS_PER_TILE)], pv.at[0])
    pltpu.sync_copy(p_hbm.at[1, pl.ds(base, ROWS_PER_TILE)], pv.at[1])

    zero_i = jnp.zeros((L,), jnp.int32)
    riota = lax.iota(jnp.int32, L)
    wregs = [w_v[pl.ds(t * L, L)] for t in range(8 * D_H // L)]

    def _wscal(k, j):
        flat = k * 8 + j
        return wregs[flat // L][flat % L]

    def _group(g, carry):
        rows = g * L + riota
        cols = [None] * D_H
        for k in range(D_H):
            a = plsc.load_gather(pv, [zero_i, rows, zero_i + k])
            b = plsc.load_gather(pv, [zero_i + 1, rows, zero_i + k])
            cols[k] = jnp.maximum(a + b, 0.0)
        for j in range(8):
            o = cols[0] * _wscal(0, j)
            for k in range(1, D_H):
                o = o + cols[k] * _wscal(k, j)
            plsc.store_scatter(x2v, [rows, zero_i + j], o)
        return carry

    lax.fori_loop(0, ROWS_PER_TILE // L, _group, 0)
    pltpu.sync_copy(x2v, xs.at[pl.ds(base, ROWS_PER_TILE)])

    _zero_acc(8, zbuf, acc, sid)
    _aggregate(8, kj, eb_hbm, out_hbm, src_v, dst_v, rows_v, xs, acc, sem,
               cid, sid, wid)


@functools.cache
def _build_spmm1(kj):
    mesh = plsc.VectorSubcoreMesh(
        core_axis_name="c", subcore_axis_name="s", num_cores=NC, num_subcores=NS
    )
    return pl.kernel(
        functools.partial(_spmm1_body, kj),
        out_type=jax.ShapeDtypeStruct((NC, N_PAD, D_H), jnp.float32),
        mesh=mesh,
        scratch_types=[
            pltpu.VMEM((kj, CHUNK), jnp.int32),
            pltpu.VMEM((kj, CHUNK), jnp.int32),
            pltpu.VMEM((2, CHUNK, D_H), jnp.float32),
            pltpu.VMEM((ROWS_PER_TILE, D_H), jnp.float32),
            pltpu.VMEM_SHARED((N_PAD, D_H), jnp.float32),
            pltpu.VMEM_SHARED((N_PAD, D_H), jnp.float32),
            pltpu.SemaphoreType.DMA,
        ],
        compiler_params=pltpu.CompilerParams(use_tc_tiling_on_sc=False),
    )


@functools.cache
def _build_spmm2(kj):
    mesh = plsc.VectorSubcoreMesh(
        core_axis_name="c", subcore_axis_name="s", num_cores=NC, num_subcores=NS
    )
    return pl.kernel(
        functools.partial(_spmm2_body, kj),
        out_type=jax.ShapeDtypeStruct((NC, N_PAD, 8), jnp.float32),
        mesh=mesh,
        scratch_types=[
            pltpu.VMEM((kj, CHUNK), jnp.int32),
            pltpu.VMEM((kj, CHUNK), jnp.int32),
            pltpu.VMEM((2, CHUNK, 8), jnp.float32),
            pltpu.VMEM((ROWS_PER_TILE, 8), jnp.float32),
            pltpu.VMEM((2, ROWS_PER_TILE, D_H), jnp.float32),
            pltpu.VMEM((ROWS_PER_TILE, 8), jnp.float32),
            pltpu.VMEM((8 * D_H,), jnp.float32),
            pltpu.VMEM_SHARED((N_PAD, 8), jnp.float32),
            pltpu.VMEM_SHARED((N_PAD, 8), jnp.float32),
            pltpu.SemaphoreType.DMA,
        ],
        compiler_params=pltpu.CompilerParams(
            use_tc_tiling_on_sc=False, needs_layout_passes=False
        ),
    )


ROWS_PER_WORKER = N_PAD // NW


def _combine_sc_body(q_hbm, out_hbm, qv, sv):
    cid = lax.axis_index("c")
    sid = lax.axis_index("s")
    wid = sid * NC + cid
    base = wid * ROWS_PER_WORKER

    pltpu.sync_copy(q_hbm.at[0, pl.ds(base, ROWS_PER_WORKER)], qv.at[0])
    pltpu.sync_copy(q_hbm.at[1, pl.ds(base, ROWS_PER_WORKER)], qv.at[1])

    zero_i = jnp.zeros((L,), jnp.int32)
    lane = lax.iota(jnp.int32, L)
    row_pat = lane // 8
    col_pat = lax.rem(lane, 8)

    def _pair(t, carry):
        rows = 2 * t + row_pat
        a = plsc.load_gather(qv, [zero_i, rows, col_pat])
        b = plsc.load_gather(qv, [zero_i + 1, rows, col_pat])
        plsc.store_scatter(sv, [rows, col_pat], a + b)
        return carry

    lax.fori_loop(0, ROWS_PER_WORKER // 2, _pair, 0)
    pltpu.sync_copy(sv, out_hbm.at[pl.ds(base, ROWS_PER_WORKER)])


@functools.cache
def _build_combine_sc():
    mesh = plsc.VectorSubcoreMesh(
        core_axis_name="c", subcore_axis_name="s", num_cores=NC, num_subcores=NS
    )
    return pl.kernel(
        _combine_sc_body,
        out_type=jax.ShapeDtypeStruct((N_PAD, 8), jnp.float32),
        mesh=mesh,
        scratch_types=[
            pltpu.VMEM((2, ROWS_PER_WORKER, 8), jnp.float32),
            pltpu.VMEM((ROWS_PER_WORKER, 8), jnp.float32),
        ],
        compiler_params=pltpu.CompilerParams(
            use_tc_tiling_on_sc=False, needs_layout_passes=False
        ),
    )


def _mm1_body(f_ref, w1_ref, o_ref):
    o_ref[...] = jnp.dot(f_ref[...], w1_ref[...],
                         preferred_element_type=jnp.float32)


def kernel(features, edge_index, W1, W2):
    e = edge_index.shape[1]
    kj = (e + NW * CHUNK - 1) // (NW * CHUNK)
    e_pad = NW * CHUNK * kj
    eb = jnp.pad(
        edge_index.astype(jnp.int32),
        ((0, 0), (0, e_pad - e)),
        constant_values=N_NODES,
    ).reshape(2, NW, kj, CHUNK)

    w2f = jnp.pad(W2, ((0, 0), (0, 8 - D_OUT))).reshape(8 * D_H)

    x1 = pl.pallas_call(
        _mm1_body,
        grid=(8,),
        in_specs=[
            pl.BlockSpec((N_PAD // 8, D_IN), lambda i: (i, 0)),
            pl.BlockSpec((D_IN, D_H), lambda i: (0, 0)),
        ],
        out_specs=pl.BlockSpec((N_PAD // 8, D_H), lambda i: (i, 0)),
        out_shape=jax.ShapeDtypeStruct((N_PAD, D_H), jnp.float32),
    )(features, W1)

    p = _build_spmm1(kj)(x1, eb)

    q = _build_spmm2(kj)(p, eb, w2f)

    out = _build_combine_sc()(q)

    return out[:N_NODES, :D_OUT]


# speedup vs baseline: 32.2034x; 1.1125x over previous
import functools

import jax
import jax.numpy as jnp
from jax import lax
from jax.experimental import pallas as pl
from jax.experimental.pallas import tpu as pltpu
from jax.experimental.pallas import tpu_sc as plsc

N_NODES = 10000
D_IN = 128
D_H = 16
D_OUT = 7

NC = 2
NS = 16
NW = NC * NS

N_PAD = 10240
CHUNK = 128
ROWS_PER_TILE = N_PAD // NS
L = 16


def _aggregate(d, kj, eb_hbm, out_hbm, src_v, dst_v, rows_v, xs, acc, sem, sem_s,
               cid, sid, wid):
    pltpu.sync_copy(eb_hbm.at[0, wid], src_v)
    pltpu.sync_copy(eb_hbm.at[1, wid], dst_v)
    plsc.subcore_barrier()

    pltpu.async_copy(xs.at[src_v.at[0]], rows_v.at[0], sem)
    pltpu.async_copy(xs.at[src_v.at[1]], rows_v.at[1], sem)

    def _chunk(j, carry):
        buf = lax.rem(j, 4)
        pltpu.make_async_copy(xs.at[src_v.at[j]], rows_v.at[buf], sem).wait()

        @pl.when(j >= 2)
        def _():
            b2 = lax.rem(j + 2, 4)
            pltpu.make_async_copy(
                rows_v.at[b2], acc.at[dst_v.at[j - 2]], sem_s
            ).wait()

        @pl.when(j + 2 < kj)
        def _():
            pltpu.async_copy(
                xs.at[src_v.at[j + 2]], rows_v.at[lax.rem(j + 2, 4)], sem
            )

        pltpu.async_copy(rows_v.at[buf], acc.at[dst_v.at[j]], sem_s, add=True)
        return carry

    lax.fori_loop(0, kj, _chunk, 0)
    for j in (kj - 2, kj - 1):
        pltpu.make_async_copy(
            rows_v.at[lax.rem(j, 4)], acc.at[dst_v.at[j]], sem_s
        ).wait()
    plsc.subcore_barrier()

    pltpu.sync_copy(
        acc.at[pl.ds(sid * ROWS_PER_TILE, ROWS_PER_TILE)],
        out_hbm.at[cid, pl.ds(sid * ROWS_PER_TILE, ROWS_PER_TILE)],
    )


def _zero_acc(d, zbuf, acc, sid):
    if d == L:
        def _zero(i, carry):
            zbuf[i, :] = jnp.zeros((L,), jnp.float32)
            return carry

        lax.fori_loop(0, ROWS_PER_TILE, _zero, 0)
    else:
        zf = jnp.zeros((L,), jnp.float32)
        zi = jnp.zeros((L,), jnp.int32)
        riota = lax.iota(jnp.int32, L)

        def _zero(g, carry):
            rows = g * L + riota
            for j in range(d):
                plsc.store_scatter(zbuf, [rows, zi + j], zf)
            return carry

        lax.fori_loop(0, ROWS_PER_TILE // L, _zero, 0)
    pltpu.sync_copy(zbuf, acc.at[pl.ds(sid * ROWS_PER_TILE, ROWS_PER_TILE)])


def _spmm1_body(kj, x_hbm, eb_hbm, out_hbm,
                src_v, dst_v, rows_v, zbuf, xs, acc, sem, sem_s):
    cid = lax.axis_index("c")
    sid = lax.axis_index("s")
    wid = sid * NC + cid

    pltpu.sync_copy(
        x_hbm.at[pl.ds(sid * ROWS_PER_TILE, ROWS_PER_TILE)],
        xs.at[pl.ds(sid * ROWS_PER_TILE, ROWS_PER_TILE)],
    )
    _zero_acc(D_H, zbuf, acc, sid)
    _aggregate(D_H, kj, eb_hbm, out_hbm, src_v, dst_v, rows_v, xs, acc, sem, sem_s,
               cid, sid, wid)


def _spmm2_body(kj, p_hbm, eb_hbm, w_hbm, out_hbm,
                src_v, dst_v, rows_v, zbuf, pv, x2v, w_v, xs, acc, sem, sem_s):
    cid = lax.axis_index("c")
    sid = lax.axis_index("s")
    wid = sid * NC + cid
    base = sid * ROWS_PER_TILE

    pltpu.sync_copy(w_hbm, w_v)
    pltpu.sync_copy(p_hbm.at[0, pl.ds(base, ROWS_PER_TILE)], pv.at[0])
    pltpu.sync_copy(p_hbm.at[1, pl.ds(base, ROWS_PER_TILE)], pv.at[1])

    zero_i = jnp.zeros((L,), jnp.int32)
    riota = lax.iota(jnp.int32, L)
    wregs = [w_v[pl.ds(t * L, L)] for t in range(8 * D_H // L)]

    def _wscal(k, j):
        flat = k * 8 + j
        return wregs[flat // L][flat % L]

    def _group(g, carry):
        rows = g * L + riota
        cols = [None] * D_H
        for k in range(D_H):
            a = plsc.load_gather(pv, [zero_i, rows, zero_i + k])
            b = plsc.load_gather(pv, [zero_i + 1, rows, zero_i + k])
            cols[k] = jnp.maximum(a + b, 0.0)
        for j in range(8):
            o = cols[0] * _wscal(0, j)
            for k in range(1, D_H):
                o = o + cols[k] * _wscal(k, j)
            plsc.store_scatter(x2v, [rows, zero_i + j], o)
        return carry

    lax.fori_loop(0, ROWS_PER_TILE // L, _group, 0)
    pltpu.sync_copy(x2v, xs.at[pl.ds(base, ROWS_PER_TILE)])

    _zero_acc(8, zbuf, acc, sid)
    _aggregate(8, kj, eb_hbm, out_hbm, src_v, dst_v, rows_v, xs, acc, sem, sem_s,
               cid, sid, wid)


@functools.cache
def _build_spmm1(kj):
    mesh = plsc.VectorSubcoreMesh(
        core_axis_name="c", subcore_axis_name="s", num_cores=NC, num_subcores=NS
    )
    return pl.kernel(
        functools.partial(_spmm1_body, kj),
        out_type=jax.ShapeDtypeStruct((NC, N_PAD, D_H), jnp.float32),
        mesh=mesh,
        scratch_types=[
            pltpu.VMEM((kj, CHUNK), jnp.int32),
            pltpu.VMEM((kj, CHUNK), jnp.int32),
            pltpu.VMEM((4, CHUNK, D_H), jnp.float32),
            pltpu.VMEM((ROWS_PER_TILE, D_H), jnp.float32),
            pltpu.VMEM_SHARED((N_PAD, D_H), jnp.float32),
            pltpu.VMEM_SHARED((N_PAD, D_H), jnp.float32),
            pltpu.SemaphoreType.DMA,
            pltpu.SemaphoreType.DMA,
        ],
        compiler_params=pltpu.CompilerParams(use_tc_tiling_on_sc=False),
    )


@functools.cache
def _build_spmm2(kj):
    mesh = plsc.VectorSubcoreMesh(
        core_axis_name="c", subcore_axis_name="s", num_cores=NC, num_subcores=NS
    )
    return pl.kernel(
        functools.partial(_spmm2_body, kj),
        out_type=jax.ShapeDtypeStruct((NC, N_PAD, 8), jnp.float32),
        mesh=mesh,
        scratch_types=[
            pltpu.VMEM((kj, CHUNK), jnp.int32),
            pltpu.VMEM((kj, CHUNK), jnp.int32),
            pltpu.VMEM((4, CHUNK, 8), jnp.float32),
            pltpu.VMEM((ROWS_PER_TILE, 8), jnp.float32),
            pltpu.VMEM((2, ROWS_PER_TILE, D_H), jnp.float32),
            pltpu.VMEM((ROWS_PER_TILE, 8), jnp.float32),
            pltpu.VMEM((8 * D_H,), jnp.float32),
            pltpu.VMEM_SHARED((N_PAD, 8), jnp.float32),
            pltpu.VMEM_SHARED((N_PAD, 8), jnp.float32),
            pltpu.SemaphoreType.DMA,
            pltpu.SemaphoreType.DMA,
        ],
        compiler_params=pltpu.CompilerParams(
            use_tc_tiling_on_sc=False, needs_layout_passes=False
        ),
    )


ROWS_PER_WORKER = N_PAD // NW


def _combine_sc_body(q_hbm, out_hbm, qv, sv):
    cid = lax.axis_index("c")
    sid = lax.axis_index("s")
    wid = sid * NC + cid
    base = wid * ROWS_PER_WORKER

    pltpu.sync_copy(q_hbm.at[0, pl.ds(base, ROWS_PER_WORKER)], qv.at[0])
    pltpu.sync_copy(q_hbm.at[1, pl.ds(base, ROWS_PER_WORKER)], qv.at[1])

    zero_i = jnp.zeros((L,), jnp.int32)
    lane = lax.iota(jnp.int32, L)
    row_pat = lane // 8
    col_pat = lax.rem(lane, 8)

    def _pair(t, carry):
        rows = 2 * t + row_pat
        a = plsc.load_gather(qv, [zero_i, rows, col_pat])
        b = plsc.load_gather(qv, [zero_i + 1, rows, col_pat])
        plsc.store_scatter(sv, [rows, col_pat], a + b)
        return carry

    lax.fori_loop(0, ROWS_PER_WORKER // 2, _pair, 0)
    pltpu.sync_copy(sv, out_hbm.at[pl.ds(base, ROWS_PER_WORKER)])


@functools.cache
def _build_combine_sc():
    mesh = plsc.VectorSubcoreMesh(
        core_axis_name="c", subcore_axis_name="s", num_cores=NC, num_subcores=NS
    )
    return pl.kernel(
        _combine_sc_body,
        out_type=jax.ShapeDtypeStruct((N_PAD, 8), jnp.float32),
        mesh=mesh,
        scratch_types=[
            pltpu.VMEM((2, ROWS_PER_WORKER, 8), jnp.float32),
            pltpu.VMEM((ROWS_PER_WORKER, 8), jnp.float32),
        ],
        compiler_params=pltpu.CompilerParams(
            use_tc_tiling_on_sc=False, needs_layout_passes=False
        ),
    )


def _mm1_body(f_ref, w1_ref, o_ref):
    o_ref[...] = jnp.dot(f_ref[...], w1_ref[...],
                         preferred_element_type=jnp.float32)


def kernel(features, edge_index, W1, W2):
    e = edge_index.shape[1]
    kj = (e + NW * CHUNK - 1) // (NW * CHUNK)
    e_pad = NW * CHUNK * kj
    eb = jnp.pad(
        edge_index.astype(jnp.int32),
        ((0, 0), (0, e_pad - e)),
        constant_values=N_NODES,
    ).reshape(2, NW, kj, CHUNK)

    w2f = jnp.pad(W2, ((0, 0), (0, 8 - D_OUT))).reshape(8 * D_H)

    x1 = pl.pallas_call(
        _mm1_body,
        grid=(2,),
        in_specs=[
            pl.BlockSpec((N_PAD // 2, D_IN), lambda i: (i, 0)),
            pl.BlockSpec((D_IN, D_H), lambda i: (0, 0)),
        ],
        out_specs=pl.BlockSpec((N_PAD // 2, D_H), lambda i: (i, 0)),
        out_shape=jax.ShapeDtypeStruct((N_PAD, D_H), jnp.float32),
    )(features, W1)

    p = _build_spmm1(kj)(x1, eb)

    q = _build_spmm2(kj)(p, eb, w2f)

    out = _build_combine_sc()(q)

    return out[:N_NODES, :D_OUT]


# speedup vs baseline: 33.9905x; 1.0555x over previous
import functools

import jax
import jax.numpy as jnp
from jax import lax
from jax.experimental import pallas as pl
from jax.experimental.pallas import tpu as pltpu
from jax.experimental.pallas import tpu_sc as plsc

N_NODES = 10000
D_IN = 128
D_H = 16
D_OUT = 7

NC = 2
NS = 16
NW = NC * NS

N_PAD = 10240
CHUNK = 128
ROWS_PER_TILE = N_PAD // NS
L = 16


def _aggregate(d, kj, eb_hbm, out_hbm, src_v, dst_v, rows_v, xs, acc, sem, sem_s,
               cid, sid, wid):
    pltpu.sync_copy(eb_hbm.at[0, wid], src_v)
    pltpu.sync_copy(eb_hbm.at[1, wid], dst_v)
    plsc.subcore_barrier()

    pltpu.async_copy(xs.at[src_v.at[0]], rows_v.at[0], sem)
    pltpu.async_copy(xs.at[src_v.at[1]], rows_v.at[1], sem)

    def _chunk(j, carry):
        buf = lax.rem(j, 4)
        pltpu.make_async_copy(xs.at[src_v.at[j]], rows_v.at[buf], sem).wait()

        @pl.when(j >= 2)
        def _():
            b2 = lax.rem(j + 2, 4)
            pltpu.make_async_copy(
                rows_v.at[b2], acc.at[dst_v.at[j - 2]], sem_s
            ).wait()

        @pl.when(j + 2 < kj)
        def _():
            pltpu.async_copy(
                xs.at[src_v.at[j + 2]], rows_v.at[lax.rem(j + 2, 4)], sem
            )

        pltpu.async_copy(rows_v.at[buf], acc.at[dst_v.at[j]], sem_s, add=True)
        return carry

    lax.fori_loop(0, kj, _chunk, 0, unroll=2)
    for j in (kj - 2, kj - 1):
        pltpu.make_async_copy(
            rows_v.at[lax.rem(j, 4)], acc.at[dst_v.at[j]], sem_s
        ).wait()
    plsc.subcore_barrier()

    pltpu.sync_copy(
        acc.at[pl.ds(sid * ROWS_PER_TILE, ROWS_PER_TILE)],
        out_hbm.at[cid, pl.ds(sid * ROWS_PER_TILE, ROWS_PER_TILE)],
    )


def _zero_acc(d, zbuf, acc, sid):
    if d == L:
        @plsc.parallel_loop(0, ROWS_PER_TILE, unroll=4)
        def _zero(i):
            zbuf[i, :] = jnp.zeros((L,), jnp.float32)
    else:
        zf = jnp.zeros((L,), jnp.float32)
        zi = jnp.zeros((L,), jnp.int32)
        riota = lax.iota(jnp.int32, L)

        @plsc.parallel_loop(0, ROWS_PER_TILE // L, unroll=2)
        def _zero(g):
            rows = g * L + riota
            for j in range(d):
                plsc.store_scatter(zbuf, [rows, zi + j], zf)
    pltpu.sync_copy(zbuf, acc.at[pl.ds(sid * ROWS_PER_TILE, ROWS_PER_TILE)])


def _spmm1_body(kj, x_hbm, eb_hbm, out_hbm,
                src_v, dst_v, rows_v, zbuf, xs, acc, sem, sem_s):
    cid = lax.axis_index("c")
    sid = lax.axis_index("s")
    wid = sid * NC + cid

    pltpu.sync_copy(
        x_hbm.at[pl.ds(sid * ROWS_PER_TILE, ROWS_PER_TILE)],
        xs.at[pl.ds(sid * ROWS_PER_TILE, ROWS_PER_TILE)],
    )
    _zero_acc(D_H, zbuf, acc, sid)
    _aggregate(D_H, kj, eb_hbm, out_hbm, src_v, dst_v, rows_v, xs, acc, sem, sem_s,
               cid, sid, wid)


def _spmm2_body(kj, p_hbm, eb_hbm, w_hbm, out_hbm,
                src_v, dst_v, rows_v, zbuf, pv, x2v, w_v, xs, acc, sem, sem_s):
    cid = lax.axis_index("c")
    sid = lax.axis_index("s")
    wid = sid * NC + cid
    base = sid * ROWS_PER_TILE

    pltpu.sync_copy(w_hbm, w_v)
    pltpu.sync_copy(p_hbm.at[0, pl.ds(base, ROWS_PER_TILE)], pv.at[0])
    pltpu.sync_copy(p_hbm.at[1, pl.ds(base, ROWS_PER_TILE)], pv.at[1])

    zero_i = jnp.zeros((L,), jnp.int32)
    riota = lax.iota(jnp.int32, L)
    wregs = [w_v[pl.ds(t * L, L)] for t in range(8 * D_H // L)]

    def _wscal(k, j):
        flat = k * 8 + j
        return wregs[flat // L][flat % L]

    @plsc.parallel_loop(0, ROWS_PER_TILE // L, unroll=2)
    def _group(g):
        rows = g * L + riota
        cols = [None] * D_H
        for k in range(D_H):
            a = plsc.load_gather(pv, [zero_i, rows, zero_i + k])
            b = plsc.load_gather(pv, [zero_i + 1, rows, zero_i + k])
            cols[k] = jnp.maximum(a + b, 0.0)
        for j in range(8):
            o = cols[0] * _wscal(0, j)
            for k in range(1, D_H):
                o = o + cols[k] * _wscal(k, j)
            plsc.store_scatter(x2v, [rows, zero_i + j], o)
    pltpu.sync_copy(x2v, xs.at[pl.ds(base, ROWS_PER_TILE)])

    _zero_acc(8, zbuf, acc, sid)
    _aggregate(8, kj, eb_hbm, out_hbm, src_v, dst_v, rows_v, xs, acc, sem, sem_s,
               cid, sid, wid)


@functools.cache
def _build_spmm1(kj):
    mesh = plsc.VectorSubcoreMesh(
        core_axis_name="c", subcore_axis_name="s", num_cores=NC, num_subcores=NS
    )
    return pl.kernel(
        functools.partial(_spmm1_body, kj),
        out_type=jax.ShapeDtypeStruct((NC, N_PAD, D_H), jnp.float32),
        mesh=mesh,
        scratch_types=[
            pltpu.VMEM((kj, CHUNK), jnp.int32),
            pltpu.VMEM((kj, CHUNK), jnp.int32),
            pltpu.VMEM((4, CHUNK, D_H), jnp.float32),
            pltpu.VMEM((ROWS_PER_TILE, D_H), jnp.float32),
            pltpu.VMEM_SHARED((N_PAD, D_H), jnp.float32),
            pltpu.VMEM_SHARED((N_PAD, D_H), jnp.float32),
            pltpu.SemaphoreType.DMA,
            pltpu.SemaphoreType.DMA,
        ],
        compiler_params=pltpu.CompilerParams(use_tc_tiling_on_sc=False),
    )


@functools.cache
def _build_spmm2(kj):
    mesh = plsc.VectorSubcoreMesh(
        core_axis_name="c", subcore_axis_name="s", num_cores=NC, num_subcores=NS
    )
    return pl.kernel(
        functools.partial(_spmm2_body, kj),
        out_type=jax.ShapeDtypeStruct((NC, N_PAD, 8), jnp.float32),
        mesh=mesh,
        scratch_types=[
            pltpu.VMEM((kj, CHUNK), jnp.int32),
            pltpu.VMEM((kj, CHUNK), jnp.int32),
            pltpu.VMEM((4, CHUNK, 8), jnp.float32),
            pltpu.VMEM((ROWS_PER_TILE, 8), jnp.float32),
            pltpu.VMEM((2, ROWS_PER_TILE, D_H), jnp.float32),
            pltpu.VMEM((ROWS_PER_TILE, 8), jnp.float32),
            pltpu.VMEM((8 * D_H,), jnp.float32),
            pltpu.VMEM_SHARED((N_PAD, 8), jnp.float32),
            pltpu.VMEM_SHARED((N_PAD, 8), jnp.float32),
            pltpu.SemaphoreType.DMA,
            pltpu.SemaphoreType.DMA,
        ],
        compiler_params=pltpu.CompilerParams(
            use_tc_tiling_on_sc=False, needs_layout_passes=False
        ),
    )


ROWS_PER_WORKER = 400
N_COMBINE_W = N_NODES // ROWS_PER_WORKER


def _combine_sc_body(q_hbm, out_hbm, qv, sv):
    cid = lax.axis_index("c")
    sid = lax.axis_index("s")
    wid = sid * NC + cid

    @pl.when(wid < N_COMBINE_W)
    def _():
        base = wid * ROWS_PER_WORKER
        pltpu.sync_copy(q_hbm.at[0, pl.ds(base, ROWS_PER_WORKER)], qv.at[0])
        pltpu.sync_copy(q_hbm.at[1, pl.ds(base, ROWS_PER_WORKER)], qv.at[1])

        zero_i = jnp.zeros((L,), jnp.int32)
        lane = lax.iota(jnp.int32, L)

        @plsc.parallel_loop(0, ROWS_PER_WORKER * D_OUT // L, unroll=2)
        def _pack(t):
            fo = t * L + lane
            r = fo // D_OUT
            c = fo - r * D_OUT
            a = plsc.load_gather(qv, [zero_i, r, c])
            b = plsc.load_gather(qv, [zero_i + 1, r, c])
            plsc.store_scatter(sv, [r, c], a + b)

        pltpu.sync_copy(sv, out_hbm.at[pl.ds(base, ROWS_PER_WORKER)])


@functools.cache
def _build_combine_sc():
    mesh = plsc.VectorSubcoreMesh(
        core_axis_name="c", subcore_axis_name="s", num_cores=NC, num_subcores=NS
    )
    return pl.kernel(
        _combine_sc_body,
        out_type=jax.ShapeDtypeStruct((N_NODES, D_OUT), jnp.float32),
        mesh=mesh,
        scratch_types=[
            pltpu.VMEM((2, ROWS_PER_WORKER, 8), jnp.float32),
            pltpu.VMEM((ROWS_PER_WORKER, D_OUT), jnp.float32),
        ],
        compiler_params=pltpu.CompilerParams(
            use_tc_tiling_on_sc=False, needs_layout_passes=False
        ),
    )


def _mm1_body(f_ref, w1_ref, o_ref):
    o_ref[...] = jnp.dot(f_ref[...], w1_ref[...],
                         preferred_element_type=jnp.float32)


def kernel(features, edge_index, W1, W2):
    e = edge_index.shape[1]
    kj = (e + NW * CHUNK - 1) // (NW * CHUNK)
    e_pad = NW * CHUNK * kj
    eb = jnp.pad(
        edge_index.astype(jnp.int32),
        ((0, 0), (0, e_pad - e)),
        constant_values=N_NODES,
    ).reshape(2, NW, kj, CHUNK)

    w2f = jnp.pad(W2, ((0, 0), (0, 8 - D_OUT))).reshape(8 * D_H)

    x1 = pl.pallas_call(
        _mm1_body,
        grid=(2,),
        in_specs=[
            pl.BlockSpec((N_PAD // 2, D_IN), lambda i: (i, 0)),
            pl.BlockSpec((D_IN, D_H), lambda i: (0, 0)),
        ],
        out_specs=pl.BlockSpec((N_PAD // 2, D_H), lambda i: (i, 0)),
        out_shape=jax.ShapeDtypeStruct((N_PAD, D_H), jnp.float32),
    )(features, W1)

    p = _build_spmm1(kj)(x1, eb)

    q = _build_spmm2(kj)(p, eb, w2f)

    return _build_combine_sc()(q)


# speedup vs baseline: 34.6983x; 1.0208x over previous
import functools

import jax
import jax.numpy as jnp
from jax import lax
from jax.experimental import pallas as pl
from jax.experimental.pallas import tpu as pltpu
from jax.experimental.pallas import tpu_sc as plsc

N_NODES = 10000
D_IN = 128
D_H = 16
D_OUT = 7

NC = 2
NS = 16
NW = NC * NS

N_PAD = 10240
CHUNK = 128
ROWS_PER_TILE = N_PAD // NS
L = 16


def _aggregate(d, kj, eb_hbm, out_hbm, src_v, dst_v, rows_v, xs, acc, sem, sem_s,
               cid, sid, wid):
    pltpu.sync_copy(eb_hbm.at[0, wid], src_v)
    pltpu.sync_copy(eb_hbm.at[1, wid], dst_v)
    plsc.subcore_barrier()

    pltpu.async_copy(xs.at[src_v.at[0]], rows_v.at[0], sem)
    pltpu.async_copy(xs.at[src_v.at[1]], rows_v.at[1], sem)

    def _chunk(j, carry):
        buf = lax.rem(j, 4)
        pltpu.make_async_copy(xs.at[src_v.at[j]], rows_v.at[buf], sem).wait()

        @pl.when(j >= 2)
        def _():
            b2 = lax.rem(j + 2, 4)
            pltpu.make_async_copy(
                rows_v.at[b2], acc.at[dst_v.at[j - 2]], sem_s
            ).wait()

        @pl.when(j + 2 < kj)
        def _():
            pltpu.async_copy(
                xs.at[src_v.at[j + 2]], rows_v.at[lax.rem(j + 2, 4)], sem
            )

        pltpu.async_copy(rows_v.at[buf], acc.at[dst_v.at[j]], sem_s, add=True)
        return carry

    lax.fori_loop(0, kj, _chunk, 0, unroll=2)
    for j in (kj - 2, kj - 1):
        pltpu.make_async_copy(
            rows_v.at[lax.rem(j, 4)], acc.at[dst_v.at[j]], sem_s
        ).wait()
    plsc.subcore_barrier()

    pltpu.sync_copy(
        acc.at[pl.ds(sid * ROWS_PER_TILE, ROWS_PER_TILE)],
        out_hbm.at[cid, pl.ds(sid * ROWS_PER_TILE, ROWS_PER_TILE)],
    )


def _zero_acc(d, zbuf, acc, sid):
    if d == L:
        @plsc.parallel_loop(0, ROWS_PER_TILE, unroll=4)
        def _zero(i):
            zbuf[i, :] = jnp.zeros((L,), jnp.float32)
    else:
        zf = jnp.zeros((L,), jnp.float32)
        zi = jnp.zeros((L,), jnp.int32)
        riota = lax.iota(jnp.int32, L)

        @plsc.parallel_loop(0, ROWS_PER_TILE // L, unroll=2)
        def _zero(g):
            rows = g * L + riota
            for j in range(d):
                plsc.store_scatter(zbuf, [rows, zi + j], zf)
    pltpu.sync_copy(zbuf, acc.at[pl.ds(sid * ROWS_PER_TILE, ROWS_PER_TILE)])


def _spmm1_body(kj, x_hbm, eb_hbm, out_hbm,
                src_v, dst_v, rows_v, zbuf, xs, acc, sem, sem_s):
    cid = lax.axis_index("c")
    sid = lax.axis_index("s")
    wid = sid * NC + cid

    pltpu.sync_copy(
        x_hbm.at[pl.ds(sid * ROWS_PER_TILE, ROWS_PER_TILE)],
        xs.at[pl.ds(sid * ROWS_PER_TILE, ROWS_PER_TILE)],
    )
    _zero_acc(D_H, zbuf, acc, sid)
    _aggregate(D_H, kj, eb_hbm, out_hbm, src_v, dst_v, rows_v, xs, acc, sem, sem_s,
               cid, sid, wid)


def _spmm2_body(kj, p_hbm, eb_hbm, w_hbm, out_hbm,
                src_v, dst_v, rows_v, zbuf, pv, x2v, w_v, xs, acc, sem, sem_s):
    cid = lax.axis_index("c")
    sid = lax.axis_index("s")
    wid = sid * NC + cid
    base = sid * ROWS_PER_TILE

    pltpu.sync_copy(w_hbm, w_v)
    pltpu.sync_copy(p_hbm.at[0, pl.ds(base, ROWS_PER_TILE)], pv.at[0])
    pltpu.sync_copy(p_hbm.at[1, pl.ds(base, ROWS_PER_TILE)], pv.at[1])

    zero_i = jnp.zeros((L,), jnp.int32)
    riota = lax.iota(jnp.int32, L)
    wregs = [w_v[pl.ds(t * L, L)] for t in range(8 * D_H // L)]

    def _wscal(k, j):
        flat = k * 8 + j
        return wregs[flat // L][flat % L]

    def _group(g, carry):
        rows = g * L + riota
        cols = [None] * D_H
        for k in range(D_H):
            a = plsc.load_gather(pv, [zero_i, rows, zero_i + k])
            b = plsc.load_gather(pv, [zero_i + 1, rows, zero_i + k])
            cols[k] = jnp.maximum(a + b, 0.0)
        for j in range(8):
            o = cols[0] * _wscal(0, j)
            for k in range(1, D_H):
                o = o + cols[k] * _wscal(k, j)
            plsc.store_scatter(x2v, [rows, zero_i + j], o)
        return carry

    lax.fori_loop(0, ROWS_PER_TILE // L, _group, 0)
    pltpu.sync_copy(x2v, xs.at[pl.ds(base, ROWS_PER_TILE)])

    _zero_acc(8, zbuf, acc, sid)
    _aggregate(8, kj, eb_hbm, out_hbm, src_v, dst_v, rows_v, xs, acc, sem, sem_s,
               cid, sid, wid)


@functools.cache
def _build_spmm1(kj):
    mesh = plsc.VectorSubcoreMesh(
        core_axis_name="c", subcore_axis_name="s", num_cores=NC, num_subcores=NS
    )
    return pl.kernel(
        functools.partial(_spmm1_body, kj),
        out_type=jax.ShapeDtypeStruct((NC, N_PAD, D_H), jnp.float32),
        mesh=mesh,
        scratch_types=[
            pltpu.VMEM((kj, CHUNK), jnp.int32),
            pltpu.VMEM((kj, CHUNK), jnp.int32),
            pltpu.VMEM((4, CHUNK, D_H), jnp.float32),
            pltpu.VMEM((ROWS_PER_TILE, D_H), jnp.float32),
            pltpu.VMEM_SHARED((N_PAD, D_H), jnp.float32),
            pltpu.VMEM_SHARED((N_PAD, D_H), jnp.float32),
            pltpu.SemaphoreType.DMA,
            pltpu.SemaphoreType.DMA,
        ],
        compiler_params=pltpu.CompilerParams(use_tc_tiling_on_sc=False),
    )


@functools.cache
def _build_spmm2(kj):
    mesh = plsc.VectorSubcoreMesh(
        core_axis_name="c", subcore_axis_name="s", num_cores=NC, num_subcores=NS
    )
    return pl.kernel(
        functools.partial(_spmm2_body, kj),
        out_type=jax.ShapeDtypeStruct((NC, N_PAD, 8), jnp.float32),
        mesh=mesh,
        scratch_types=[
            pltpu.VMEM((kj, CHUNK), jnp.int32),
            pltpu.VMEM((kj, CHUNK), jnp.int32),
            pltpu.VMEM((4, CHUNK, 8), jnp.float32),
            pltpu.VMEM((ROWS_PER_TILE, 8), jnp.float32),
            pltpu.VMEM((2, ROWS_PER_TILE, D_H), jnp.float32),
            pltpu.VMEM((ROWS_PER_TILE, 8), jnp.float32),
            pltpu.VMEM((8 * D_H,), jnp.float32),
            pltpu.VMEM_SHARED((N_PAD, 8), jnp.float32),
            pltpu.VMEM_SHARED((N_PAD, 8), jnp.float32),
            pltpu.SemaphoreType.DMA,
            pltpu.SemaphoreType.DMA,
        ],
        compiler_params=pltpu.CompilerParams(
            use_tc_tiling_on_sc=False, needs_layout_passes=False
        ),
    )


ROWS_PER_WORKER = 400
N_COMBINE_W = N_NODES // ROWS_PER_WORKER


def _combine_sc_body(q_hbm, out_hbm, qv, sv):
    cid = lax.axis_index("c")
    sid = lax.axis_index("s")
    wid = sid * NC + cid

    @pl.when(wid < N_COMBINE_W)
    def _():
        base = wid * ROWS_PER_WORKER
        pltpu.sync_copy(q_hbm.at[0, pl.ds(base, ROWS_PER_WORKER)], qv.at[0])
        pltpu.sync_copy(q_hbm.at[1, pl.ds(base, ROWS_PER_WORKER)], qv.at[1])

        zero_i = jnp.zeros((L,), jnp.int32)
        lane = lax.iota(jnp.int32, L)

        @plsc.parallel_loop(0, ROWS_PER_WORKER * D_OUT // L, unroll=2)
        def _pack(t):
            fo = t * L + lane
            r = fo // D_OUT
            c = fo - r * D_OUT
            a = plsc.load_gather(qv, [zero_i, r, c])
            b = plsc.load_gather(qv, [zero_i + 1, r, c])
            plsc.store_scatter(sv, [r, c], a + b)

        pltpu.sync_copy(sv, out_hbm.at[pl.ds(base, ROWS_PER_WORKER)])


@functools.cache
def _build_combine_sc():
    mesh = plsc.VectorSubcoreMesh(
        core_axis_name="c", subcore_axis_name="s", num_cores=NC, num_subcores=NS
    )
    return pl.kernel(
        _combine_sc_body,
        out_type=jax.ShapeDtypeStruct((N_NODES, D_OUT), jnp.float32),
        mesh=mesh,
        scratch_types=[
            pltpu.VMEM((2, ROWS_PER_WORKER, 8), jnp.float32),
            pltpu.VMEM((ROWS_PER_WORKER, D_OUT), jnp.float32),
        ],
        compiler_params=pltpu.CompilerParams(
            use_tc_tiling_on_sc=False, needs_layout_passes=False
        ),
    )


def _mm1_body(f_ref, w1_ref, o_ref):
    o_ref[...] = jnp.dot(f_ref[...], w1_ref[...],
                         preferred_element_type=jnp.float32)


def kernel(features, edge_index, W1, W2):
    e = edge_index.shape[1]
    kj = (e + NW * CHUNK - 1) // (NW * CHUNK)
    e_pad = NW * CHUNK * kj
    eb = jnp.pad(
        edge_index.astype(jnp.int32),
        ((0, 0), (0, e_pad - e)),
        constant_values=N_NODES,
    ).reshape(2, NW, kj, CHUNK)

    w2f = jnp.pad(W2, ((0, 0), (0, 8 - D_OUT))).reshape(8 * D_H)

    x1 = pl.pallas_call(
        _mm1_body,
        grid=(2,),
        in_specs=[
            pl.BlockSpec((N_PAD // 2, D_IN), lambda i: (i, 0)),
            pl.BlockSpec((D_IN, D_H), lambda i: (0, 0)),
        ],
        out_specs=pl.BlockSpec((N_PAD // 2, D_H), lambda i: (i, 0)),
        out_shape=jax.ShapeDtypeStruct((N_PAD, D_H), jnp.float32),
    )(features, W1)

    p = _build_spmm1(kj)(x1, eb)

    q = _build_spmm2(kj)(p, eb, w2f)

    return _build_combine_sc()(q)
